# Initial kernel scaffold; baseline (speedup 1.0000x reference)
import jax
import jax.numpy as jnp
from jax.experimental import pallas as pl


def kernel(x, edge_index, edge_values, W):
    raise NotImplementedError("write your pallas kernel here")


# speedup vs baseline: 4.6684x; 4.6684x over previous
import functools

import jax
import jax.numpy as jnp
from jax import lax
from jax.experimental import pallas as pl
from jax.experimental.pallas import tpu as pltpu
from jax.experimental.pallas import tpu_sc as plsc

N = 10000
E = 320000
D = 128
NC = 2
NS = 16
NW = NC * NS
C = 128
NCHUNK = E // C
NPAD = 10240
ROWS_PER_TILE = NPAD // NS


def _sc_matvec_body(h_hbm, row_hbm, col_hbm, ev_hbm, out_hbm,
                    acc, gbuf, zbuf, cidx, ridx, evb, sem):
    c = lax.axis_index("c")
    s = lax.axis_index("s")
    wid = s * NC + c

    def zfill(i, _):
        for g in range(8):
            zbuf[i, pl.ds(g * 16, 16)] = jnp.zeros((16,), jnp.float32)
        return 0
    lax.fori_loop(0, 128, zfill, 0)
    r0 = s * ROWS_PER_TILE
    for t in range(5):
        pltpu.sync_copy(zbuf.at[pl.ds(0, 128)],
                        acc.at[pl.ds(r0 + t * 128, 128)])
    plsc.subcore_barrier()

    nloc = 78 + jnp.where(wid < NCHUNK % NW, 1, 0)

    def chunk_body(i, _):
        j = wid + i * NW
        base = j * C
        pltpu.sync_copy(col_hbm.at[pl.ds(base, C)], cidx)
        pltpu.sync_copy(row_hbm.at[pl.ds(base, C)], ridx)
        pltpu.sync_copy(ev_hbm.at[pl.ds(base, C)], evb)
        pltpu.async_copy(h_hbm.at[cidx], gbuf, sem).wait()

        def scale(b, _):
            ev_vec = evb[pl.ds(b * 16, 16)]
            for l in range(16):
                bval = jnp.full((16,), ev_vec[l], jnp.float32)
                e = b * 16 + l
                for g in range(8):
                    sl = (e, pl.ds(g * 16, 16))
                    gbuf[sl] = gbuf[sl] * bval
            return 0
        lax.fori_loop(0, C // 16, scale, 0)

        pltpu.sync_copy(gbuf, acc.at[ridx], add=True)
        return 0
    lax.fori_loop(0, nloc, chunk_body, 0)

    plsc.subcore_barrier()

    for t in range(5):
        pltpu.sync_copy(acc.at[pl.ds(r0 + t * 128, 128)],
                        out_hbm.at[c, pl.ds(r0 + t * 128, 128)])


_sc_matvec = pl.kernel(
    _sc_matvec_body,
    out_type=jax.ShapeDtypeStruct((NC, NPAD, D), jnp.float32),
    mesh=plsc.VectorSubcoreMesh(core_axis_name="c", subcore_axis_name="s",
                                num_cores=NC, num_subcores=NS),
    scratch_types=[
        pltpu.VMEM_SHARED((NPAD, D), jnp.float32),
        pltpu.VMEM((C, D), jnp.float32),
        pltpu.VMEM((128, D), jnp.float32),
        pltpu.VMEM((C,), jnp.int32),
        pltpu.VMEM((C,), jnp.int32),
        pltpu.VMEM((C,), jnp.float32),
        pltpu.SemaphoreType.DMA,
    ],
)


def _tc_add_body(a_ref, b_ref, o_ref):
    o_ref[...] = a_ref[...] + b_ref[...]


def _tc_final_body(x_ref, h1_ref, p0_ref, p1_ref, w0_ref, w1_ref, w2_ref,
                   o_ref):
    h2 = p0_ref[...] + p1_ref[...]
    o_ref[...] = (
        jnp.dot(x_ref[...], w0_ref[...], preferred_element_type=jnp.float32)
        + jnp.dot(h1_ref[...], w1_ref[...], preferred_element_type=jnp.float32)
        + jnp.dot(h2, w2_ref[...], preferred_element_type=jnp.float32))


_RB = 1000


def _tc_add(a, b):
    return pl.pallas_call(
        _tc_add_body,
        grid=(N // _RB,),
        in_specs=[pl.BlockSpec((_RB, D), lambda i: (i, 0))] * 2,
        out_specs=pl.BlockSpec((_RB, D), lambda i: (i, 0)),
        out_shape=jax.ShapeDtypeStruct((N, D), jnp.float32),
    )(a, b)


def _tc_final(x, h1, p0, p1, w0, w1, w2):
    wspec = pl.BlockSpec((D, D), lambda i: (0, 0))
    rspec = pl.BlockSpec((_RB, D), lambda i: (i, 0))
    return pl.pallas_call(
        _tc_final_body,
        grid=(N // _RB,),
        in_specs=[rspec, rspec, rspec, rspec, wspec, wspec, wspec],
        out_specs=rspec,
        out_shape=jax.ShapeDtypeStruct((N, D), jnp.float32),
    )(x, h1, p0, p1, w0, w1, w2)


@jax.jit
def kernel(x, edge_index, edge_values, W):
    row = edge_index[0]
    col = edge_index[1]
    w0 = W[:, :, 0]
    w1 = W[:, :, 1]
    w2 = W[:, :, 2]
    p1 = _sc_matvec(x, row, col, edge_values)
    h1 = _tc_add(p1[0, :N], p1[1, :N])
    p2 = _sc_matvec(h1, row, col, edge_values)
    return _tc_final(x, h1, p2[0, :N], p2[1, :N], w0, w1, w2)
